# Initial kernel scaffold; baseline (speedup 1.0000x reference)
import jax
import jax.numpy as jnp
from jax.experimental import pallas as pl


def kernel(x, tables):
    raise NotImplementedError("write your pallas kernel here")


# speedup vs baseline: 1.0535x; 1.0535x over previous
import functools

import jax
import jax.numpy as jnp
from jax import lax
from jax.experimental import pallas as pl
from jax.experimental.pallas import tpu as pltpu
from jax.experimental.pallas import tpu_sc as plsc

B = 16384
F = 26
VOCAB = 100000
NW = 32
BPW = B // NW
CHUNK = 128
NCH = BPW // CHUNK
ROWS = F * NCH


def _sc_body(idx_hbm, tab_hbm, out_hbm, idx_v, vals_v, acc_v, sem):
    cid = lax.axis_index("c")
    sid = lax.axis_index("s")
    wid = sid * 2 + cid
    base = wid * BPW

    pltpu.sync_copy(idx_hbm.at[wid], idx_v)

    def zero_body(j, carry):
        acc_v[pl.ds(j * 16, 16)] = jnp.zeros((16,), jnp.float32)
        return carry

    lax.fori_loop(0, BPW // 16, zero_body, 0, unroll=8)

    def row_body(r, carry):
        off = ((r // NCH) * VOCAB).astype(jnp.int32)
        offv = jnp.broadcast_to(off, (16,))
        for s in range(CHUNK // 16):
            sl = pl.ds(s * 16, 16)
            idx_v[r, sl] = idx_v[r, sl] + offv
        pltpu.async_copy(tab_hbm.at[idx_v.at[r]], vals_v, sem).wait()
        accb = (r % NCH) * CHUNK
        for s in range(CHUNK // 16):
            asl = pl.ds(accb + s * 16, 16)
            acc_v[asl] = acc_v[asl] + vals_v[pl.ds(s * 16, 16)]
        return carry

    lax.fori_loop(0, ROWS, row_body, 0)

    pltpu.sync_copy(acc_v, out_hbm.at[pl.ds(base, BPW)])


@functools.partial(
    pl.kernel,
    mesh=plsc.VectorSubcoreMesh(core_axis_name="c", subcore_axis_name="s"),
    out_type=jax.ShapeDtypeStruct((B,), jnp.float32),
    scratch_types=[
        pltpu.VMEM((ROWS, CHUNK), jnp.int32),
        pltpu.VMEM((CHUNK,), jnp.float32),
        pltpu.VMEM((BPW,), jnp.float32),
        pltpu.SemaphoreType.DMA,
    ],
)
def _sc_call(idx_hbm, tab_hbm, out_hbm, idx_v, vals_v, acc_v, sem):
    _sc_body(idx_hbm, tab_hbm, out_hbm, idx_v, vals_v, acc_v, sem)


@jax.jit
def kernel(x, tables):
    idx = (
        x.astype(jnp.int32)
        .T.reshape(F, NW, NCH, CHUNK)
        .transpose(1, 0, 2, 3)
        .reshape(NW, ROWS, CHUNK)
    )
    tab = tables.reshape(-1)
    out = _sc_call(idx, tab)
    return out.reshape(B, 1)


# speedup vs baseline: 1.4587x; 1.3847x over previous
import functools

import jax
import jax.numpy as jnp
from jax import lax
from jax.experimental import pallas as pl
from jax.experimental.pallas import tpu as pltpu
from jax.experimental.pallas import tpu_sc as plsc

B = 16384
F = 26
VOCAB = 100000
NW = 32
BPW = B // NW
CHUNK = 128
NCH = BPW // CHUNK
ROWS = F * NCH


def _sc_body(idx_hbm, tab_hbm, out_hbm, idx_v, vals_v, acc_v, sem):
    cid = lax.axis_index("c")
    sid = lax.axis_index("s")
    wid = sid * 2 + cid
    base = wid * BPW

    pltpu.sync_copy(idx_hbm.at[wid], idx_v)

    def zero_body(j, carry):
        acc_v[pl.ds(j * 16, 16)] = jnp.zeros((16,), jnp.float32)
        return carry

    lax.fori_loop(0, BPW // 16, zero_body, 0, unroll=8)

    def issue_body(r, carry):
        off = ((r // NCH) * VOCAB).astype(jnp.int32)
        offv = jnp.broadcast_to(off, (16,))
        for s in range(CHUNK // 16):
            sl = pl.ds(s * 16, 16)
            idx_v[r, sl] = idx_v[r, sl] + offv
        pltpu.async_copy(tab_hbm.at[idx_v.at[r]], vals_v.at[r], sem)
        return carry

    lax.fori_loop(0, ROWS, issue_body, 0)

    def drain_body(r, carry):
        pltpu.make_async_copy(tab_hbm.at[idx_v.at[0]], vals_v.at[0], sem).wait()
        return carry

    lax.fori_loop(0, ROWS, drain_body, 0)

    def acc_body(r, carry):
        accb = (r % NCH) * CHUNK
        for s in range(CHUNK // 16):
            asl = pl.ds(accb + s * 16, 16)
            acc_v[asl] = acc_v[asl] + vals_v[r, pl.ds(s * 16, 16)]
        return carry

    lax.fori_loop(0, ROWS, acc_body, 0)

    pltpu.sync_copy(acc_v, out_hbm.at[pl.ds(base, BPW)])


@functools.partial(
    pl.kernel,
    mesh=plsc.VectorSubcoreMesh(core_axis_name="c", subcore_axis_name="s"),
    out_type=jax.ShapeDtypeStruct((B,), jnp.float32),
    scratch_types=[
        pltpu.VMEM((ROWS, CHUNK), jnp.int32),
        pltpu.VMEM((ROWS, CHUNK), jnp.float32),
        pltpu.VMEM((BPW,), jnp.float32),
        pltpu.SemaphoreType.DMA,
    ],
)
def _sc_call(idx_hbm, tab_hbm, out_hbm, idx_v, vals_v, acc_v, sem):
    _sc_body(idx_hbm, tab_hbm, out_hbm, idx_v, vals_v, acc_v, sem)


@jax.jit
def kernel(x, tables):
    idx = (
        x.astype(jnp.int32)
        .T.reshape(F, NW, NCH, CHUNK)
        .transpose(1, 0, 2, 3)
        .reshape(NW, ROWS, CHUNK)
    )
    tab = tables.reshape(-1)
    out = _sc_call(idx, tab)
    return out.reshape(B, 1)


# speedup vs baseline: 4.7334x; 3.2450x over previous
import functools

import jax
import jax.numpy as jnp
from jax import lax
from jax.experimental import pallas as pl
from jax.experimental.pallas import tpu as pltpu
from jax.experimental.pallas import tpu_sc as plsc

B = 16384
F = 26
VOCAB = 100000
NC = 2
NS = 16
HALF = B // NC
AROWS = HALF // 128


def _sc_body(xt_hbm, tab_hbm, out_hbm, tab_loc, xcol_v, vals_v, idx64_v,
             acc_sh, sem_t, sem_x):
    cid = lax.axis_index("c")
    sid = lax.axis_index("s")

    for k in range(AROWS // 16):
        idx64_v[pl.ds(k * 16, 16)] = lax.iota(jnp.int32, 16) + jnp.broadcast_to(
            jnp.int32(k * 16), (16,)
        )

    @pl.when(sid == 0)
    def _():
        def zrow(r, carry):
            for s2 in range(8):
                vals_v[r, pl.ds(s2 * 16, 16)] = jnp.zeros((16,), jnp.float32)
            return carry

        lax.fori_loop(0, AROWS, zrow, 0)
        pltpu.sync_copy(vals_v, acc_sh)

    plsc.subcore_barrier()

    def do_field(f, accumulate):
        cp_t = pltpu.async_copy(tab_hbm.at[f], tab_loc, sem_t)
        cp_x = pltpu.async_copy(
            xt_hbm.at[pl.ds(f * B + cid * HALF, HALF)], xcol_v, sem_x
        )
        cp_x.wait()
        cp_t.wait()

        def grow(r, carry):
            for s2 in range(8):
                sl = pl.ds(s2 * 16, 16)
                idx16 = xcol_v[pl.ds(r * 128 + s2 * 16, 16)]
                v16 = plsc.load_gather(tab_loc, [idx16])
                if accumulate:
                    vals_v[r, sl] = vals_v[r, sl] + v16
                else:
                    vals_v[r, sl] = v16
            return carry

        lax.fori_loop(0, AROWS, grow, 0)

    do_field(sid, accumulate=False)

    @pl.when(sid < F - NS)
    def _():
        do_field(sid + NS, accumulate=True)

    pltpu.sync_copy(vals_v, acc_sh.at[idx64_v], add=True)
    plsc.subcore_barrier()

    @pl.when(sid == 0)
    def _():
        pltpu.sync_copy(acc_sh, out_hbm.at[cid])


@functools.partial(
    pl.kernel,
    mesh=plsc.VectorSubcoreMesh(core_axis_name="c", subcore_axis_name="s"),
    out_type=jax.ShapeDtypeStruct((NC, AROWS, 128), jnp.float32),
    scratch_types=[
        pltpu.VMEM((VOCAB,), jnp.float32),
        pltpu.VMEM((HALF,), jnp.int32),
        pltpu.VMEM((AROWS, 128), jnp.float32),
        pltpu.VMEM((AROWS,), jnp.int32),
        pltpu.VMEM_SHARED((AROWS, 128), jnp.float32),
        pltpu.SemaphoreType.DMA,
        pltpu.SemaphoreType.DMA,
    ],
    compiler_params=pltpu.CompilerParams(needs_layout_passes=False),
)
def _sc_call(xt_hbm, tab_hbm, out_hbm, tab_loc, xcol_v, vals_v, idx64_v,
             acc_sh, sem_t, sem_x):
    _sc_body(xt_hbm, tab_hbm, out_hbm, tab_loc, xcol_v, vals_v, idx64_v,
             acc_sh, sem_t, sem_x)


@jax.jit
def kernel(x, tables):
    xt = x.astype(jnp.int32).T.reshape(-1)
    out = _sc_call(xt, tables.reshape(F, VOCAB))
    return out.reshape(B, 1)


# speedup vs baseline: 4.7375x; 1.0009x over previous
import functools

import jax
import jax.numpy as jnp
from jax import lax
from jax.experimental import pallas as pl
from jax.experimental.pallas import tpu as pltpu
from jax.experimental.pallas import tpu_sc as plsc

B = 16384
F = 26
VOCAB = 100000
NC = 2
NS = 16
HALF = B // NC
AROWS = HALF // 128


def _sc_body(xt_hbm, tab_hbm, out_hbm, tab_loc, xcol_v, vals_v, idx64_v,
             acc_sh, sem_t, sem_x):
    cid = lax.axis_index("c")
    sid = lax.axis_index("s")

    for k in range(AROWS // 16):
        idx64_v[pl.ds(k * 16, 16)] = lax.iota(jnp.int32, 16) + jnp.broadcast_to(
            jnp.int32(k * 16), (16,)
        )

    @pl.when(sid == 0)
    def _():
        def zrow(r, carry):
            for s2 in range(8):
                vals_v[r, pl.ds(s2 * 16, 16)] = jnp.zeros((16,), jnp.float32)
            return carry

        lax.fori_loop(0, AROWS, zrow, 0)
        pltpu.sync_copy(vals_v, acc_sh)

    plsc.subcore_barrier()

    def do_field(f, accumulate):
        cp_t = pltpu.async_copy(tab_hbm.at[f], tab_loc, sem_t)
        cp_x = pltpu.async_copy(
            xt_hbm.at[pl.ds(f * B + cid * HALF, HALF)], xcol_v, sem_x
        )
        cp_x.wait()
        cp_t.wait()

        def grow(r, carry):
            for s2 in range(8):
                sl = pl.ds(s2 * 16, 16)
                idx16 = xcol_v[pl.ds(r * 128 + s2 * 16, 16)]
                v16 = plsc.load_gather(tab_loc, [idx16])
                if accumulate:
                    vals_v[r, sl] = vals_v[r, sl] + v16
                else:
                    vals_v[r, sl] = v16
            return carry

        lax.fori_loop(0, AROWS, grow, 0)

    do_field(sid, accumulate=False)

    @pl.when(sid < F - NS)
    def _():
        do_field(sid + NS, accumulate=True)

    pltpu.sync_copy(vals_v, acc_sh.at[idx64_v], add=True)
    plsc.subcore_barrier()

    @pl.when(sid == 0)
    def _():
        pltpu.sync_copy(acc_sh, out_hbm.at[cid])


@functools.partial(
    pl.kernel,
    mesh=plsc.VectorSubcoreMesh(core_axis_name="c", subcore_axis_name="s"),
    out_type=jax.ShapeDtypeStruct((NC, AROWS, 128), jnp.float32),
    scratch_types=[
        pltpu.VMEM((VOCAB,), jnp.float32),
        pltpu.VMEM((HALF,), jnp.int32),
        pltpu.VMEM((AROWS, 128), jnp.float32),
        pltpu.VMEM((AROWS,), jnp.int32),
        pltpu.VMEM_SHARED((AROWS, 128), jnp.float32),
        pltpu.SemaphoreType.DMA,
        pltpu.SemaphoreType.DMA,
    ],
    compiler_params=pltpu.CompilerParams(needs_layout_passes=False, use_tc_tiling_on_sc=True),
)
def _sc_call(xt_hbm, tab_hbm, out_hbm, tab_loc, xcol_v, vals_v, idx64_v,
             acc_sh, sem_t, sem_x):
    _sc_body(xt_hbm, tab_hbm, out_hbm, tab_loc, xcol_v, vals_v, idx64_v,
             acc_sh, sem_t, sem_x)


@jax.jit
def kernel(x, tables):
    xt = x.astype(jnp.int32).T.reshape(-1)
    out = _sc_call(xt, tables.reshape(F, VOCAB))
    return out.reshape(B, 1)
